# Initial kernel scaffold; baseline (speedup 1.0000x reference)
import jax
import jax.numpy as jnp
from jax.experimental import pallas as pl


def kernel(z, bank, ptr):
    raise NotImplementedError("write your pallas kernel here")


# speedup vs baseline: 1.9565x; 1.9565x over previous
import functools

import jax
import jax.numpy as jnp
from jax import lax
from jax.experimental import pallas as pl
from jax.experimental.pallas import tpu as pltpu
from jax.experimental.pallas import tpu_sc as plsc

SIZE = 1000000
DIM = 32
BATCH = 16384

GROUP = 128 // DIM
GSIZE = SIZE // GROUP
GBATCH = BATCH // GROUP

COPY_ROWS = 2000
NUM_BLOCKS = GSIZE // COPY_ROWS

NUM_CORES = 2
NUM_SUBCORES = 16
NW = NUM_CORES * NUM_SUBCORES
GROUPS_PER_W = GBATCH // NW


def _tc_copy_normalize(z_ref, bank_ref, zn_ref, out_ref):
  i = pl.program_id(0)
  out_ref[...] = bank_ref[...]

  @pl.when(i == 0)
  def _():
    z = z_ref[...]
    norm = jnp.sqrt(jnp.sum(z * z, axis=1, keepdims=True))
    zn_ref[...] = z / jnp.clip(norm, 1e-12)


def _tc_stage(z, bank_g):
  zn, out_g = pl.pallas_call(
      _tc_copy_normalize,
      grid=(NUM_BLOCKS,),
      in_specs=[
          pl.BlockSpec((BATCH, DIM), lambda i: (0, 0)),
          pl.BlockSpec((COPY_ROWS, 128), lambda i: (i, 0)),
      ],
      out_specs=[
          pl.BlockSpec((BATCH, DIM), lambda i: (0, 0)),
          pl.BlockSpec((COPY_ROWS, 128), lambda i: (i, 0)),
      ],
      out_shape=[
          jax.ShapeDtypeStruct((BATCH, DIM), jnp.float32),
          jax.ShapeDtypeStruct((GSIZE, 128), jnp.float32),
      ],
      name="bank_copy_normalize",
  )(z, bank_g)
  return zn, out_g


def _sc_scatter_body(zn_hbm, idx_hbm, bank_ref, rows_v, idx_v, sem):
  wid = lax.axis_index("s") * NUM_CORES + lax.axis_index("c")
  pltpu.sync_copy(zn_hbm.at[pl.ds(wid * GROUPS_PER_W, GROUPS_PER_W)], rows_v)
  pltpu.sync_copy(idx_hbm.at[pl.ds(wid, 1)], idx_v)
  pltpu.async_copy(rows_v, bank_ref.at[idx_v.at[0]], sem).wait()


@functools.cache
def _sc_scatter():
  mesh = plsc.VectorSubcoreMesh(
      core_axis_name="c", subcore_axis_name="s",
      num_cores=NUM_CORES, num_subcores=NUM_SUBCORES)
  return pl.kernel(
      _sc_scatter_body,
      out_type=(),
      mesh=mesh,
      scratch_types=[
          pltpu.VMEM((GROUPS_PER_W, 128), jnp.float32),
          pltpu.VMEM((1, GROUPS_PER_W), jnp.int32),
          pltpu.SemaphoreType.DMA,
      ],
      name="bank_scatter_sc",
  )


def kernel(z, bank, ptr):
  bank_g = bank.reshape(GSIZE, 128)
  zn, out_g = _tc_stage(z, bank_g)
  zn_g = zn.reshape(GBATCH, 128)
  p = ptr[0]
  gidx = (p // GROUP + jnp.arange(GBATCH, dtype=jnp.int32)) % GSIZE
  idx2d = gidx.reshape(NW, GROUPS_PER_W)
  ref = jax.new_ref(out_g)
  _sc_scatter()(zn_g, idx2d, ref)
  new_bank = ref[...].reshape(SIZE, DIM)
  new_ptr = (p + BATCH) % SIZE
  wrapped = jnp.logical_or(new_ptr < p, p + BATCH >= SIZE)
  return new_bank, jnp.array([new_ptr], dtype=jnp.int32), jnp.reshape(wrapped, (1,))
